# Initial kernel scaffold; baseline (speedup 1.0000x reference)
import jax
import jax.numpy as jnp
from jax.experimental import pallas as pl


def kernel(xyz, features, W, b):
    raise NotImplementedError("write your pallas kernel here")


# speedup vs baseline: 2.7513x; 2.7513x over previous
import functools

import jax
import jax.numpy as jnp
from jax.experimental import pallas as pl


def _fps_proj_body(xyzp_ref, xyz4_ref, feats_ref, w68_ref, np_ref, pf_ref, q_ref,
                   *, npoint):
    xp = xyzp_ref[0]
    x = xp[0]
    y = xp[1]
    z = xp[2]
    sub = x.shape[0]
    lane = x.shape[1]
    n = sub * lane
    iota = (jax.lax.broadcasted_iota(jnp.int32, (sub, lane), 0) * lane
            + jax.lax.broadcasted_iota(jnp.int32, (sub, lane), 1))

    def extract(v, idx):
        return jnp.sum(jnp.where(iota == idx, v, 0.0))

    px0 = extract(x, 0)
    py0 = extract(y, 0)
    pz0 = extract(z, 0)
    d0 = (x - px0) ** 2 + (y - py0) ** 2 + (z - pz0) ** 2
    np_ref[0, 0:1, :] = xyz4_ref[0, 0:1, :]

    def step(i, d_min):
        m = jnp.max(d_min)
        idx = jnp.min(jnp.where(d_min == m, iota, jnp.int32(n)))
        px = extract(x, idx)
        py = extract(y, idx)
        pz = extract(z, idx)
        d_new = (x - px) ** 2 + (y - py) ** 2 + (z - pz) ** 2
        np_ref[0, pl.ds(i, 1), :] = xyz4_ref[0, pl.ds(idx, 1), :]
        return jnp.minimum(d_min, d_new)

    jax.lax.fori_loop(1, npoint, step, d0)

    g = jnp.concatenate([xyz4_ref[0], feats_ref[0]], axis=1)
    pf_ref[0] = jax.lax.dot_general(g, w68_ref[...],
                                    (((1,), (0,)), ((), ())),
                                    preferred_element_type=jnp.float32)
    q_ref[0] = jax.lax.dot_general(np_ref[0], w68_ref[0:4, :],
                                   (((1,), (0,)), ((), ())),
                                   preferred_element_type=jnp.float32)


def _knn_pool_body(np8_ref, xyzt_ref, pf_ref, q_ref, b_ref, out_ref,
                   *, nsample):
    np8 = np8_ref[0]
    xt = xyzt_ref[0]
    n = xt.shape[1]
    s = jax.lax.dot_general(np8, xt, (((1,), (0,)), ((), ())),
                            preferred_element_type=jnp.float32)
    sqx = jnp.sum(xt * xt, axis=0, keepdims=True)
    sqn = jnp.sum(np8 * np8, axis=1, keepdims=True)
    d = (sqn + sqx) - 2.0 * s
    col = jax.lax.broadcasted_iota(jnp.int32, (8, n), 1)
    row8 = jax.lax.broadcasted_iota(jnp.int32, (8, 1), 0)
    neg = jnp.float32(-jnp.inf)
    acc0 = tuple(jnp.full((1, 64), neg, jnp.float32) for _ in range(8))

    def step(_, carry):
        d, accs = carry
        m = jnp.min(d, axis=1, keepdims=True)
        idxs = jnp.min(jnp.where(d == m, col, jnp.int32(n)),
                       axis=1, keepdims=True)
        d = jnp.where(col == idxs, jnp.float32(jnp.inf), d)
        new_accs = []
        for r in range(8):
            ir = jnp.sum(jnp.where(row8 == r, idxs, 0))
            rowv = pf_ref[0, pl.ds(ir, 1), :]
            new_accs.append(jnp.maximum(accs[r], rowv))
        return d, tuple(new_accs)

    _, accs = jax.lax.fori_loop(0, nsample, step, (d, acc0))
    pooled = jnp.concatenate(accs, axis=0)
    out_ref[0] = jnp.maximum(pooled - q_ref[0] + b_ref[...], 0.0)


def _impl(xyz, features, w, bias, npoint, nsample):
    bsz, n, _ = xyz.shape
    c = features.shape[1]
    xyz_t = jnp.transpose(xyz, (0, 2, 1))
    xyz_planes = xyz_t.reshape(bsz, 3, 8, n // 8)
    xyz4 = jnp.concatenate([xyz, jnp.zeros((bsz, n, 1), jnp.float32)], axis=2)
    xyz4_t = jnp.transpose(xyz4, (0, 2, 1))
    feats_t = jnp.transpose(features, (0, 2, 1))
    w_t = jnp.transpose(w)
    w68 = jnp.concatenate([w_t[:3], jnp.zeros((1, c), jnp.float32), w_t[3:]],
                          axis=0)
    b2 = bias.reshape(1, c)

    fps = pl.pallas_call(
        functools.partial(_fps_proj_body, npoint=npoint),
        grid=(bsz,),
        in_specs=[
            pl.BlockSpec((1, 3, 8, n // 8), lambda i: (i, 0, 0, 0)),
            pl.BlockSpec((1, n, 4), lambda i: (i, 0, 0)),
            pl.BlockSpec((1, n, c), lambda i: (i, 0, 0)),
            pl.BlockSpec((c + 4, c), lambda i: (0, 0)),
        ],
        out_specs=[
            pl.BlockSpec((1, npoint, 4), lambda i: (i, 0, 0)),
            pl.BlockSpec((1, n, c), lambda i: (i, 0, 0)),
            pl.BlockSpec((1, npoint, c), lambda i: (i, 0, 0)),
        ],
        out_shape=[
            jax.ShapeDtypeStruct((bsz, npoint, 4), jnp.float32),
            jax.ShapeDtypeStruct((bsz, n, c), jnp.float32),
            jax.ShapeDtypeStruct((bsz, npoint, c), jnp.float32),
        ],
    )
    new_pts4, pf, q = fps(xyz_planes, xyz4, feats_t, w68)

    ngroups = npoint // 8
    pooled = pl.pallas_call(
        functools.partial(_knn_pool_body, nsample=nsample),
        grid=(bsz, ngroups),
        in_specs=[
            pl.BlockSpec((1, 8, 4), lambda i, g: (i, g, 0)),
            pl.BlockSpec((1, 4, n), lambda i, g: (i, 0, 0)),
            pl.BlockSpec((1, n, c), lambda i, g: (i, 0, 0)),
            pl.BlockSpec((1, 8, c), lambda i, g: (i, g, 0)),
            pl.BlockSpec((1, c), lambda i, g: (0, 0)),
        ],
        out_specs=pl.BlockSpec((1, 8, c), lambda i, g: (i, g, 0)),
        out_shape=jax.ShapeDtypeStruct((bsz, npoint, c), jnp.float32),
    )(new_pts4, xyz4_t, pf, q, b2)

    new_xyz = new_pts4[:, :, :3]
    new_features = jnp.transpose(pooled, (0, 2, 1))
    return new_xyz, new_features


def kernel(xyz, features, W, b):
    return _impl(xyz, features, W, b, 512, 32)
